# Initial kernel scaffold; baseline (speedup 1.0000x reference)
import jax
import jax.numpy as jnp
from jax.experimental import pallas as pl


def kernel(peptide_x, peptide_esm_x, mhc_x, peptide_emb, mhc_emb):
    raise NotImplementedError("write your pallas kernel here")


# speedup vs baseline: 1.5100x; 1.5100x over previous
import functools
import jax
import jax.numpy as jnp
from jax import lax
from jax.experimental import pallas as pl
from jax.experimental.pallas import tpu as pltpu
from jax.experimental.pallas import tpu_sc as plsc

B = 4096
PEP_LEN = 21
MHC_LEN = 34
CORE_LEN = 15
EMB = 128
PAD = 3
VOCAB = 30

NC = 2
NS = 16
NW = NC * NS

ROWS_W = B // NW
PEP_W = ROWS_W * PEP_LEN
MHC_W = ROWS_W * MHC_LEN
MSK_W = ROWS_W * CORE_LEN
CHUNK = 128
PEP_CHUNKS = PEP_W // CHUNK
MHC_CHUNKS = MHC_W // CHUNK
NBUF = 4
MSK_VECS = MSK_W // 16

_mesh = plsc.VectorSubcoreMesh(core_axis_name="c", subcore_axis_name="s")


@functools.partial(
    pl.kernel,
    mesh=_mesh,
    out_type=[
        jax.ShapeDtypeStruct((B * PEP_LEN, EMB), jnp.float32),
        jax.ShapeDtypeStruct((B * MHC_LEN, EMB), jnp.float32),
        jax.ShapeDtypeStruct((B * CORE_LEN,), jnp.int32),
    ],
    scratch_types=[
        pltpu.VMEM((PEP_W,), jnp.int32),
        pltpu.VMEM((MHC_W,), jnp.int32),
        pltpu.VMEM((MSK_W + 16,), jnp.int32),
    ]
    + [pltpu.VMEM((CHUNK, EMB), jnp.float32) for _ in range(NBUF)]
    + [pltpu.SemaphoreType.DMA for _ in range(2 * NBUF + 1)],
)
def _emb_lookup(pep_idx_hbm, mhc_idx_hbm, pep_tab, mhc_tab,
                pep_out, mhc_out, msk_out,
                pep_idx_v, mhc_idx_v, msk_v,
                buf0, buf1, buf2, buf3,
                gsem0, gsem1, gsem2, gsem3,
                osem0, osem1, osem2, osem3, msem):
    bufs = [buf0, buf1, buf2, buf3]
    gsems = [gsem0, gsem1, gsem2, gsem3]
    osems = [osem0, osem1, osem2, osem3]

    wid = lax.axis_index("s") * NC + lax.axis_index("c")

    pltpu.sync_copy(pep_idx_hbm.at[pl.ds(wid * PEP_W, PEP_W)], pep_idx_v)
    pltpu.sync_copy(mhc_idx_hbm.at[pl.ds(wid * MHC_W, MHC_W)], mhc_idx_v)

    def prime(idx_v, tab, nchunks):
        g = [None] * NBUF
        for s in range(min(NBUF, nchunks)):
            g[s] = pltpu.async_copy(
                tab.at[idx_v.at[pl.ds(s * CHUNK, CHUNK)]], bufs[s], gsems[s])
        return g

    def drain(g, idx_v, tab, out, base, nchunks):
        o = [None] * NBUF
        for j in range(nchunks):
            s = j % NBUF
            g[s].wait()
            o[s] = pltpu.async_copy(
                bufs[s], out.at[pl.ds(base + j * CHUNK, CHUNK)], osems[s])
            nxt = j + NBUF
            if nxt < nchunks:
                o[s].wait()
                g[s] = pltpu.async_copy(
                    tab.at[idx_v.at[pl.ds(nxt * CHUNK, CHUNK)]], bufs[s],
                    gsems[s])
        for j in range(max(0, nchunks - NBUF), nchunks):
            o[j % NBUF].wait()

    g = prime(pep_idx_v, pep_tab, PEP_CHUNKS)

    def mask_body(b, carry):
        vals = pep_idx_v[pl.ds(b * PEP_LEN + PAD, 16)]
        m = jnp.where(vals != jnp.zeros((16,), jnp.int32),
                      jnp.ones((16,), jnp.int32),
                      jnp.zeros((16,), jnp.int32))
        msk_v[pl.ds(b * CORE_LEN, 16)] = m
        return carry

    lax.fori_loop(0, ROWS_W, mask_body, 0)
    mcopy = pltpu.async_copy(msk_v.at[pl.ds(0, MSK_W)],
                             msk_out.at[pl.ds(wid * MSK_W, MSK_W)], msem)

    drain(g, pep_idx_v, pep_tab, pep_out, wid * PEP_W, PEP_CHUNKS)

    g = prime(mhc_idx_v, mhc_tab, MHC_CHUNKS)
    drain(g, mhc_idx_v, mhc_tab, mhc_out, wid * MHC_W, MHC_CHUNKS)

    mcopy.wait()


def kernel(peptide_x, peptide_esm_x, mhc_x, peptide_emb, mhc_emb):
    del peptide_esm_x
    pep_idx = peptide_x.astype(jnp.int32).reshape(B * PEP_LEN)
    mhc_idx = mhc_x.astype(jnp.int32).reshape(B * MHC_LEN)
    pep_out, mhc_out, msk = _emb_lookup(pep_idx, mhc_idx,
                                        peptide_emb, mhc_emb)
    peptide_out = pep_out.reshape(B, PEP_LEN, EMB)
    mhc_out = mhc_out.reshape(B, MHC_LEN, EMB)
    masks = msk.reshape(B, CORE_LEN).astype(jnp.bool_)
    return (peptide_out, masks, mhc_out)


# speedup vs baseline: 1.5117x; 1.0011x over previous
import functools
import jax
import jax.numpy as jnp
from jax import lax
from jax.experimental import pallas as pl
from jax.experimental.pallas import tpu as pltpu
from jax.experimental.pallas import tpu_sc as plsc

B = 4096
PEP_LEN = 21
MHC_LEN = 34
CORE_LEN = 15
EMB = 128
PAD = 3
VOCAB = 30

NC = 2
NS = 16
NW = NC * NS

ROWS_W = B // NW
PEP_W = ROWS_W * PEP_LEN
MHC_W = ROWS_W * MHC_LEN
MSK_W = ROWS_W * CORE_LEN
CHUNK = 128
PEP_CHUNKS = PEP_W // CHUNK
MHC_CHUNKS = MHC_W // CHUNK
NBUF = 6
PREF = 4
MSK_VECS = MSK_W // 16

_mesh = plsc.VectorSubcoreMesh(core_axis_name="c", subcore_axis_name="s")


@functools.partial(
    pl.kernel,
    mesh=_mesh,
    out_type=[
        jax.ShapeDtypeStruct((B * PEP_LEN, EMB), jnp.float32),
        jax.ShapeDtypeStruct((B * MHC_LEN, EMB), jnp.float32),
        jax.ShapeDtypeStruct((B * CORE_LEN,), jnp.int32),
    ],
    scratch_types=[
        pltpu.VMEM((PEP_W,), jnp.int32),
        pltpu.VMEM((MHC_W,), jnp.int32),
        pltpu.VMEM((MSK_W + 16,), jnp.int32),
    ]
    + [pltpu.VMEM((CHUNK, EMB), jnp.float32) for _ in range(NBUF)]
    + [pltpu.SemaphoreType.DMA for _ in range(2 * NBUF + 1)],
)
def _emb_lookup(pep_idx_hbm, mhc_idx_hbm, pep_tab, mhc_tab,
                pep_out, mhc_out, msk_out,
                pep_idx_v, mhc_idx_v, msk_v, *bufs_and_sems):
    bufs = list(bufs_and_sems[:NBUF])
    gsems = list(bufs_and_sems[NBUF:2 * NBUF])
    osems = list(bufs_and_sems[2 * NBUF:3 * NBUF])
    msem = bufs_and_sems[3 * NBUF]

    wid = lax.axis_index("s") * NC + lax.axis_index("c")

    pltpu.sync_copy(pep_idx_hbm.at[pl.ds(wid * PEP_W, PEP_W)], pep_idx_v)
    pltpu.sync_copy(mhc_idx_hbm.at[pl.ds(wid * MHC_W, MHC_W)], mhc_idx_v)

    def gather(idx_v, tab, c):
        s = c % NBUF
        return pltpu.async_copy(
            tab.at[idx_v.at[pl.ds(c * CHUNK, CHUNK)]], bufs[s], gsems[s])

    def prime(idx_v, tab, nchunks):
        g = [None] * NBUF
        for c in range(min(PREF, nchunks)):
            g[c % NBUF] = gather(idx_v, tab, c)
        return g

    def drain(g, idx_v, tab, out, base, nchunks):
        o = [None] * NBUF
        pending = [False] * NBUF
        for j in range(nchunks):
            s = j % NBUF
            g[s].wait()
            o[s] = pltpu.async_copy(
                bufs[s], out.at[pl.ds(base + j * CHUNK, CHUNK)], osems[s])
            pending[s] = True
            c = j + PREF
            if c < nchunks:
                sc = c % NBUF
                if pending[sc]:
                    o[sc].wait()
                    pending[sc] = False
                g[sc] = gather(idx_v, tab, c)
        for s in range(NBUF):
            if pending[s]:
                o[s].wait()

    g = prime(pep_idx_v, pep_tab, PEP_CHUNKS)

    def mask_body(b, carry):
        vals = pep_idx_v[pl.ds(b * PEP_LEN + PAD, 16)]
        m = jnp.where(vals != jnp.zeros((16,), jnp.int32),
                      jnp.ones((16,), jnp.int32),
                      jnp.zeros((16,), jnp.int32))
        msk_v[pl.ds(b * CORE_LEN, 16)] = m
        return carry

    lax.fori_loop(0, ROWS_W, mask_body, 0)
    mcopy = pltpu.async_copy(msk_v.at[pl.ds(0, MSK_W)],
                             msk_out.at[pl.ds(wid * MSK_W, MSK_W)], msem)

    drain(g, pep_idx_v, pep_tab, pep_out, wid * PEP_W, PEP_CHUNKS)

    g = prime(mhc_idx_v, mhc_tab, MHC_CHUNKS)
    drain(g, mhc_idx_v, mhc_tab, mhc_out, wid * MHC_W, MHC_CHUNKS)

    mcopy.wait()


def kernel(peptide_x, peptide_esm_x, mhc_x, peptide_emb, mhc_emb):
    del peptide_esm_x
    pep_idx = peptide_x.astype(jnp.int32).reshape(B * PEP_LEN)
    mhc_idx = mhc_x.astype(jnp.int32).reshape(B * MHC_LEN)
    pep_out, mhc_out, msk = _emb_lookup(pep_idx, mhc_idx,
                                        peptide_emb, mhc_emb)
    peptide_out = pep_out.reshape(B, PEP_LEN, EMB)
    mhc_out = mhc_out.reshape(B, MHC_LEN, EMB)
    masks = msk.reshape(B, CORE_LEN).astype(jnp.bool_)
    return (peptide_out, masks, mhc_out)


# speedup vs baseline: 3.7524x; 2.4822x over previous
import functools
import jax
import jax.numpy as jnp
from jax import lax
from jax.experimental import pallas as pl
from jax.experimental.pallas import tpu as pltpu
from jax.experimental.pallas import tpu_sc as plsc

B = 4096
PEP_LEN = 21
MHC_LEN = 34
CORE_LEN = 15
EMB = 128
PAD = 3
VOCAB = 30

NC = 2
NS = 16
NW = NC * NS

ROWS_W = B // NW
PEP_W = ROWS_W * PEP_LEN
MHC_W = ROWS_W * MHC_LEN
MSK_W = ROWS_W * CORE_LEN
CHUNK = 128
PEP_CHUNKS = PEP_W // CHUNK
MHC_CHUNKS = MHC_W // CHUNK
NBUF = 6
PREF = 4
MSK_VECS = MSK_W // 16

_mesh = plsc.VectorSubcoreMesh(core_axis_name="c", subcore_axis_name="s")


@functools.partial(
    pl.kernel,
    mesh=_mesh,
    out_type=[
        jax.ShapeDtypeStruct((B * PEP_LEN, EMB), jnp.float32),
        jax.ShapeDtypeStruct((B * MHC_LEN, EMB), jnp.float32),
        jax.ShapeDtypeStruct((B * CORE_LEN,), jnp.int32),
    ],
    scratch_types=[
        pltpu.VMEM((PEP_W,), jnp.int32),
        pltpu.VMEM((MHC_W,), jnp.int32),
        pltpu.VMEM((MSK_W + 16,), jnp.int32),
        pltpu.VMEM_SHARED((VOCAB, EMB), jnp.float32),
        pltpu.VMEM_SHARED((VOCAB, EMB), jnp.float32),
    ]
    + [pltpu.VMEM((CHUNK, EMB), jnp.float32) for _ in range(NBUF)]
    + [pltpu.SemaphoreType.DMA for _ in range(2 * NBUF + 1)],
)
def _emb_lookup(pep_idx_hbm, mhc_idx_hbm, pep_tab, mhc_tab,
                pep_out, mhc_out, msk_out,
                pep_idx_v, mhc_idx_v, msk_v, pep_tab_v, mhc_tab_v,
                *bufs_and_sems):
    bufs = list(bufs_and_sems[:NBUF])
    gsems = list(bufs_and_sems[NBUF:2 * NBUF])
    osems = list(bufs_and_sems[2 * NBUF:3 * NBUF])
    msem = bufs_and_sems[3 * NBUF]

    wid = lax.axis_index("s") * NC + lax.axis_index("c")

    pltpu.sync_copy(pep_idx_hbm.at[pl.ds(wid * PEP_W, PEP_W)], pep_idx_v)
    pltpu.sync_copy(mhc_idx_hbm.at[pl.ds(wid * MHC_W, MHC_W)], mhc_idx_v)

    @pl.when(lax.axis_index("s") == 0)
    def _stage_tables():
        pltpu.sync_copy(pep_tab, pep_tab_v)
        pltpu.sync_copy(mhc_tab, mhc_tab_v)

    plsc.subcore_barrier()

    def gather(idx_v, tab, c):
        s = c % NBUF
        return pltpu.async_copy(
            tab.at[idx_v.at[pl.ds(c * CHUNK, CHUNK)]], bufs[s], gsems[s])

    def prime(idx_v, tab, nchunks):
        g = [None] * NBUF
        for c in range(min(PREF, nchunks)):
            g[c % NBUF] = gather(idx_v, tab, c)
        return g

    def drain(g, idx_v, tab, out, base, nchunks):
        o = [None] * NBUF
        pending = [False] * NBUF
        for j in range(nchunks):
            s = j % NBUF
            if g[s] is not None:
                g[s].wait()
            o[s] = pltpu.async_copy(
                bufs[s], out.at[pl.ds(base + j * CHUNK, CHUNK)], osems[s])
            pending[s] = True
            c = j + PREF
            if c < nchunks:
                sc = c % NBUF
                if pending[sc]:
                    o[sc].wait()
                    pending[sc] = False
                g[sc] = gather(idx_v, tab, c)
        for s in range(NBUF):
            if pending[s]:
                o[s].wait()

    g = prime(pep_idx_v, pep_tab_v, PEP_CHUNKS)

    def mask_body(b, carry):
        vals = pep_idx_v[pl.ds(b * PEP_LEN + PAD, 16)]
        m = jnp.where(vals != jnp.zeros((16,), jnp.int32),
                      jnp.ones((16,), jnp.int32),
                      jnp.zeros((16,), jnp.int32))
        msk_v[pl.ds(b * CORE_LEN, 16)] = m
        return carry

    lax.fori_loop(0, ROWS_W, mask_body, 0)
    mcopy = pltpu.async_copy(msk_v.at[pl.ds(0, MSK_W)],
                             msk_out.at[pl.ds(wid * MSK_W, MSK_W)], msem)

    drain(g, pep_idx_v, pep_tab_v, pep_out, wid * PEP_W, PEP_CHUNKS)

    g = prime(mhc_idx_v, mhc_tab_v, MHC_CHUNKS)
    drain(g, mhc_idx_v, mhc_tab_v, mhc_out, wid * MHC_W, MHC_CHUNKS)

    mcopy.wait()


def kernel(peptide_x, peptide_esm_x, mhc_x, peptide_emb, mhc_emb):
    del peptide_esm_x
    pep_idx = peptide_x.astype(jnp.int32).reshape(B * PEP_LEN)
    mhc_idx = mhc_x.astype(jnp.int32).reshape(B * MHC_LEN)
    pep_out, mhc_out, msk = _emb_lookup(pep_idx, mhc_idx,
                                        peptide_emb, mhc_emb)
    peptide_out = pep_out.reshape(B, PEP_LEN, EMB)
    mhc_out = mhc_out.reshape(B, MHC_LEN, EMB)
    masks = msk.reshape(B, CORE_LEN).astype(jnp.bool_)
    return (peptide_out, masks, mhc_out)
